# baseline (speedup 1.0000x reference)
import functools

import jax
import jax.numpy as jnp
from jax import lax
from jax.experimental import pallas as pl
from jax.experimental.pallas import tpu as pltpu
from jax.experimental.pallas import tpu_sc as plsc

_NUM_K = 64
_ROWS, _COLS = 2048, 512
_NC, _NS = 2, 16
_NW = _NC * _NS
_LANES = 16

_SC_ROWS = 256
_TC_ROWS = _ROWS - _SC_ROWS
_ROWS_W = _SC_ROWS // _NW
_TC_KBLK = 4

_mesh = plsc.VectorSubcoreMesh(
    core_axis_name="c", subcore_axis_name="s", num_cores=_NC, num_subcores=_NS
)


@functools.partial(
    pl.kernel,
    mesh=_mesh,
    out_type=jax.ShapeDtypeStruct((_SC_ROWS, _COLS), jnp.float32),
    scratch_types=[
        pltpu.VMEM((_ROWS_W, _COLS), jnp.float32),
        pltpu.VMEM((_ROWS_W, _COLS), jnp.float32),
        pltpu.VMEM((_ROWS_W, _COLS), jnp.float32),
        pltpu.SemaphoreType.DMA,
        pltpu.SemaphoreType.DMA,
        pltpu.SemaphoreType.DMA,
    ],
)
def _gaines_or_sc(in_hbm, out_hbm, acc, buf0, buf1, sem_a, sem0, sem1):
    wid = lax.axis_index("s") * _NC + lax.axis_index("c")
    row0 = _TC_ROWS + wid * _ROWS_W

    bufs = (buf0, buf1)
    sems = (sem0, sem1)

    cp_acc = pltpu.async_copy(
        in_hbm.at[0, pl.ds(row0, _ROWS_W), :], acc, sem_a
    )
    pending = {}
    for k in (1, 2):
        pending[k] = pltpu.async_copy(
            in_hbm.at[k, pl.ds(row0, _ROWS_W), :], bufs[k % 2], sems[k % 2]
        )
    cp_acc.wait()

    for k in range(1, _NUM_K):
        b = bufs[k % 2]
        pending[k].wait()

        @plsc.parallel_loop(0, _ROWS_W, 1)
        def _accum(r):
            @plsc.parallel_loop(0, _COLS, _LANES, unroll=8)
            def _accum_row(c):
                cc = pl.multiple_of(c, _LANES)
                plsc.addupdate(
                    acc.at[r, pl.ds(cc, _LANES)], b[r, pl.ds(cc, _LANES)]
                )

        nxt = k + 2
        if nxt < _NUM_K:
            pending[nxt] = pltpu.async_copy(
                in_hbm.at[nxt, pl.ds(row0, _ROWS_W), :], b, sems[k % 2]
            )

    @plsc.parallel_loop(0, _ROWS_W, 1)
    def _threshold(r):
        @plsc.parallel_loop(0, _COLS, _LANES, unroll=8)
        def _threshold_row(c):
            cc = pl.multiple_of(c, _LANES)
            v = acc[r, pl.ds(cc, _LANES)]
            acc[r, pl.ds(cc, _LANES)] = jnp.where(v > 0.0, 1.0, 0.0).astype(
                jnp.float32
            )

    pltpu.sync_copy(acc, out_hbm.at[pl.ds(wid * _ROWS_W, _ROWS_W), :])


def _tc_body(x_ref, o_ref):
    part = jnp.max(x_ref[...], axis=0)
    k = pl.program_id(0)

    @pl.when(k == 0)
    def _init():
        o_ref[0:_TC_ROWS, :] = part

    @pl.when(k > 0)
    def _accum():
        o_ref[0:_TC_ROWS, :] = jnp.maximum(o_ref[0:_TC_ROWS, :], part)


_gaines_or_tc = pl.pallas_call(
    _tc_body,
    grid=(_NUM_K // _TC_KBLK,),
    in_specs=[
        pl.BlockSpec((_TC_KBLK, _TC_ROWS, _COLS), lambda k: (k, 0, 0)),
    ],
    out_specs=pl.BlockSpec((_ROWS, _COLS), lambda k: (0, 0)),
    out_shape=jax.ShapeDtypeStruct((_ROWS, _COLS), jnp.float32),
)


def kernel(input):
    sc_out = _gaines_or_sc(input)
    tc_out = _gaines_or_tc(input)
    return lax.dynamic_update_slice(tc_out, sc_out, (_TC_ROWS, 0))


# speedup vs baseline: 1.2236x; 1.2236x over previous
import functools

import jax
import jax.numpy as jnp
from jax import lax
from jax.experimental import pallas as pl
from jax.experimental.pallas import tpu as pltpu
from jax.experimental.pallas import tpu_sc as plsc

_NUM_K = 64
_ROWS, _COLS = 2048, 512
_NC, _NS = 2, 16
_NW = _NC * _NS
_LANES = 16

_SC_ROWS = 256
_TC_ROWS = _ROWS - _SC_ROWS
_ROWS_W = _SC_ROWS // _NW
_TC_KBLK = 4

_mesh = plsc.VectorSubcoreMesh(
    core_axis_name="c", subcore_axis_name="s", num_cores=_NC, num_subcores=_NS
)


@functools.partial(
    pl.kernel,
    mesh=_mesh,
    out_type=jax.ShapeDtypeStruct((_SC_ROWS, _COLS), jnp.float32),
    scratch_types=[
        pltpu.VMEM((_ROWS_W, _COLS), jnp.float32),
        pltpu.VMEM((_ROWS_W, _COLS), jnp.float32),
        pltpu.VMEM((_ROWS_W, _COLS), jnp.float32),
        pltpu.SemaphoreType.DMA,
        pltpu.SemaphoreType.DMA,
        pltpu.SemaphoreType.DMA,
    ],
)
def _gaines_or_sc(in_hbm, out_hbm, acc, buf0, buf1, sem_a, sem0, sem1):
    wid = lax.axis_index("s") * _NC + lax.axis_index("c")
    row0 = _TC_ROWS + wid * _ROWS_W

    bufs = (buf0, buf1)
    sems = (sem0, sem1)

    cp_acc = pltpu.async_copy(
        in_hbm.at[0, pl.ds(row0, _ROWS_W), :], acc, sem_a
    )
    pending = {}
    for k in (1, 2):
        pending[k] = pltpu.async_copy(
            in_hbm.at[k, pl.ds(row0, _ROWS_W), :], bufs[k % 2], sems[k % 2]
        )
    cp_acc.wait()

    for k in range(1, _NUM_K):
        b = bufs[k % 2]
        pending[k].wait()

        @plsc.parallel_loop(0, _ROWS_W, 1)
        def _accum(r):
            @plsc.parallel_loop(0, _COLS, _LANES, unroll=8)
            def _accum_row(c):
                cc = pl.multiple_of(c, _LANES)
                plsc.addupdate(
                    acc.at[r, pl.ds(cc, _LANES)], b[r, pl.ds(cc, _LANES)]
                )

        nxt = k + 2
        if nxt < _NUM_K:
            pending[nxt] = pltpu.async_copy(
                in_hbm.at[nxt, pl.ds(row0, _ROWS_W), :], b, sems[k % 2]
            )

    @plsc.parallel_loop(0, _ROWS_W, 1)
    def _threshold(r):
        @plsc.parallel_loop(0, _COLS, _LANES, unroll=8)
        def _threshold_row(c):
            cc = pl.multiple_of(c, _LANES)
            v = acc[r, pl.ds(cc, _LANES)]
            acc[r, pl.ds(cc, _LANES)] = jnp.where(v > 0.0, 1.0, 0.0).astype(
                jnp.float32
            )

    pltpu.sync_copy(acc, out_hbm.at[pl.ds(wid * _ROWS_W, _ROWS_W), :])


def _tc_body(x_ref, o_ref):
    part = jnp.max(x_ref[...], axis=0)
    k = pl.program_id(1)

    @pl.when(k == 0)
    def _init():
        o_ref[...] = part

    @pl.when(k > 0)
    def _accum():
        o_ref[...] = jnp.maximum(o_ref[...], part)


_RSPLIT = 2
_RBLK = _ROWS // _RSPLIT

_gaines_or_tc = pl.pallas_call(
    _tc_body,
    grid=(_RSPLIT, _NUM_K // _TC_KBLK),
    in_specs=[
        pl.BlockSpec((_TC_KBLK, _RBLK, _COLS), lambda r, k: (k, r, 0)),
    ],
    out_specs=pl.BlockSpec((_RBLK, _COLS), lambda r, k: (r, 0)),
    out_shape=jax.ShapeDtypeStruct((_ROWS, _COLS), jnp.float32),
)


def kernel(input):
    return _gaines_or_tc(input)
